# baseline (speedup 1.0000x reference)
import functools

import jax
import jax.numpy as jnp
from jax import lax
from jax.experimental import pallas as pl
from jax.experimental.pallas import tpu as pltpu
from jax.experimental.pallas import tpu_sc as plsc

NC = 2
NS = 16
NW = NC * NS
LANES = 16
BE = 128
NBUF = 4
PADR = 8


def _row_part(n_nodes):
    rbase = (n_nodes // (NS * 8)) * 8
    rem = n_nodes - rbase * NS
    return rbase, rem


@functools.lru_cache(maxsize=None)
def _make_deg(n_nodes, nctot):
    nch = nctot // NW
    njb = nch // NBUF
    rbase, rem = _row_part(n_nodes)
    mesh = plsc.VectorSubcoreMesh(core_axis_name="c", subcore_axis_name="s")

    @functools.partial(
        pl.kernel,
        mesh=mesh,
        compiler_params=pltpu.CompilerParams(use_tc_tiling_on_sc=False),
        out_type=jax.ShapeDtypeStruct((NC, n_nodes, LANES), jnp.float32),
        scratch_types=[
            pltpu.VMEM((nch, BE), jnp.int32),
            pltpu.VMEM((BE, LANES), jnp.float32),
            pltpu.VMEM_SHARED((n_nodes + PADR, LANES), jnp.float32),
            pltpu.SemaphoreType.DMA((NBUF,)),
        ],
    )
    def deg_kernel(eidx_hbm, zeros_hbm, ones_hbm, out_hbm,
                   dst_v, ones_v, acc, dsem):
        c = lax.axis_index("c")
        s = lax.axis_index("s")
        wid = s * NC + c
        pltpu.sync_copy(zeros_hbm.at[pl.ds(s * rbase, rbase)],
                        acc.at[pl.ds(s * rbase, rbase)])
        if rem:
            @pl.when(s == 0)
            def _():
                pltpu.sync_copy(zeros_hbm.at[pl.ds(rbase * NS, rem)],
                                acc.at[pl.ds(rbase * NS, rem)])
        pltpu.sync_copy(ones_hbm, ones_v)
        pltpu.sync_copy(eidx_hbm.at[1, pl.ds(wid * nch, nch)], dst_v)
        plsc.subcore_barrier()

        def scat(i, b):
            pltpu.async_copy(ones_v, acc.at[dst_v.at[i]], dsem.at[b],
                             add=True)

        def scat_wait(i, b):
            pltpu.make_async_copy(ones_v, acc.at[dst_v.at[i]],
                                  dsem.at[b]).wait()

        for b in range(NBUF):
            scat(b, b)

        def body(j, carry):
            i0 = j * NBUF
            for b in range(NBUF):
                scat_wait(i0 + b, b)
                scat(i0 + NBUF + b, b)
            return carry

        lax.fori_loop(0, njb - 1, body, 0)
        i0 = (njb - 1) * NBUF
        for b in range(NBUF):
            scat_wait(i0 + b, b)
        plsc.subcore_barrier()
        pltpu.sync_copy(acc.at[pl.ds(s * rbase, rbase)],
                        out_hbm.at[c, pl.ds(s * rbase, rbase)])
        if rem:
            @pl.when(s == 0)
            def _():
                pltpu.sync_copy(acc.at[pl.ds(rbase * NS, rem)],
                                out_hbm.at[c, pl.ds(rbase * NS, rem)])

    return deg_kernel


@functools.lru_cache(maxsize=None)
def _make_agg(n_nodes, nctot):
    nch = nctot // NW
    njb = nch // NBUF
    rbase, rem = _row_part(n_nodes)
    mesh = plsc.VectorSubcoreMesh(core_axis_name="c", subcore_axis_name="s")

    @functools.partial(
        pl.kernel,
        mesh=mesh,
        compiler_params=pltpu.CompilerParams(use_tc_tiling_on_sc=False),
        out_type=jax.ShapeDtypeStruct((NC, n_nodes, LANES), jnp.float32),
        scratch_types=[
            pltpu.VMEM((nch, BE), jnp.int32),
            pltpu.VMEM((nch, BE), jnp.int32),
            pltpu.VMEM((NBUF, BE, LANES), jnp.float32),
            pltpu.VMEM_SHARED((n_nodes + PADR, LANES), jnp.float32),
            pltpu.SemaphoreType.DMA((NBUF,)),
            pltpu.SemaphoreType.DMA((NBUF,)),
        ],
    )
    def agg_kernel(table_hbm, eidx_hbm, zeros_hbm, out_hbm,
                   src_v, dst_v, rows_v, acc, gsem, ssem):
        c = lax.axis_index("c")
        s = lax.axis_index("s")
        wid = s * NC + c
        pltpu.sync_copy(zeros_hbm.at[pl.ds(s * rbase, rbase)],
                        acc.at[pl.ds(s * rbase, rbase)])
        if rem:
            @pl.when(s == 0)
            def _():
                pltpu.sync_copy(zeros_hbm.at[pl.ds(rbase * NS, rem)],
                                acc.at[pl.ds(rbase * NS, rem)])
        pltpu.sync_copy(eidx_hbm.at[0, pl.ds(wid * nch, nch)], src_v)
        pltpu.sync_copy(eidx_hbm.at[1, pl.ds(wid * nch, nch)], dst_v)
        plsc.subcore_barrier()

        def gather(i, b):
            pltpu.async_copy(table_hbm.at[src_v.at[i]], rows_v.at[b],
                             gsem.at[b])

        def gather_wait(i, b):
            pltpu.make_async_copy(table_hbm.at[src_v.at[i]], rows_v.at[b],
                                  gsem.at[b]).wait()

        def scat(i, b):
            pltpu.async_copy(rows_v.at[b], acc.at[dst_v.at[i]], ssem.at[b],
                             add=True)

        def scat_wait(i, b):
            pltpu.make_async_copy(rows_v.at[b], acc.at[dst_v.at[i]],
                                  ssem.at[b]).wait()

        for b in range(NBUF):
            gather(b, b)

        def body(j, carry):
            i0 = j * NBUF
            for b in range(NBUF):
                gather_wait(i0 + b, b)
                scat(i0 + b, b)
            for b in range(NBUF):
                scat_wait(i0 + b, b)
                gather(i0 + NBUF + b, b)
            return carry

        lax.fori_loop(0, njb - 1, body, 0)
        i0 = (njb - 1) * NBUF
        for b in range(NBUF):
            gather_wait(i0 + b, b)
            scat(i0 + b, b)
        for b in range(NBUF):
            scat_wait(i0 + b, b)
        plsc.subcore_barrier()
        pltpu.sync_copy(acc.at[pl.ds(s * rbase, rbase)],
                        out_hbm.at[c, pl.ds(s * rbase, rbase)])
        if rem:
            @pl.when(s == 0)
            def _():
                pltpu.sync_copy(acc.at[pl.ds(rbase * NS, rem)],
                                out_hbm.at[c, pl.ds(rbase * NS, rem)])

    return agg_kernel


def _tc_pre(x_ref, w1_ref, b1_ref, degp_ref,
            xs_ref, selfb1_ref, dis_ref, inv_ref):
    xw = jnp.dot(x_ref[...], w1_ref[...], preferred_element_type=jnp.float32)
    deg = degp_ref[0] + degp_ref[1] + 1.0
    dis = lax.rsqrt(deg)
    inv = 1.0 / deg
    xs_ref[...] = xw * dis
    selfb1_ref[...] = xw * inv + b1_ref[...]
    dis_ref[...] = dis
    inv_ref[...] = inv


def _tc_mid(s1p_ref, selfb1_ref, dis_ref, inv_ref, w2p_ref, b2p_ref,
            hs_ref, self2_ref):
    s1 = s1p_ref[0] + s1p_ref[1]
    h = jnp.maximum(dis_ref[...] * s1 + selfb1_ref[...], 0.0)
    hw = jnp.dot(h, w2p_ref[...], preferred_element_type=jnp.float32)
    hs_ref[...] = hw * dis_ref[...]
    self2_ref[...] = hw * inv_ref[...] + b2p_ref[...]


def _tc_post(s2p_ref, self2_ref, dis_ref, out_ref, *, d_out):
    o = dis_ref[...] * (s2p_ref[0] + s2p_ref[1]) + self2_ref[...]
    col = lax.broadcasted_iota(jnp.int32, o.shape, 1)
    om = jnp.where(col < d_out, o, -jnp.inf)
    m = jnp.max(om, axis=1, keepdims=True)
    e = jnp.exp(om - m)
    ssum = jnp.sum(e, axis=1, keepdims=True)
    out_ref[...] = (o - m - jnp.log(ssum))[:, :d_out]


BN = 1000


def _rows(n, width):
    return pl.BlockSpec((BN, width), lambda i: (i, 0))


def _part(n):
    return pl.BlockSpec((NC, BN, LANES), lambda i: (0, i, 0))


def _full(*shape):
    return pl.BlockSpec(shape, lambda i: (0,) * len(shape))


def kernel(x, edge_index, W1, b1, W2, b2):
    n, d_in = x.shape
    d_hid = W1.shape[1]
    d_out = W2.shape[1]
    n_edges = edge_index.shape[1]
    assert d_hid == LANES and d_out <= LANES
    assert n % NS == 0 and n % BN == 0
    grid = (n // BN,)

    nctot = -(-n_edges // (NW * BE * NBUF)) * NW * NBUF
    e_pad = nctot * BE - n_edges
    pad = jnp.concatenate(
        [jnp.zeros((1, e_pad), jnp.int32),
         jnp.full((1, e_pad), n, jnp.int32)], axis=0)
    eidx = jnp.concatenate([edge_index, pad], axis=1).reshape(2, nctot, BE)

    zeros = jnp.zeros((n, LANES), jnp.float32)
    ones = jnp.ones((BE, LANES), jnp.float32)
    w2p = jnp.zeros((LANES, LANES), jnp.float32).at[:, :d_out].set(W2)
    b2p = jnp.zeros((LANES,), jnp.float32).at[:d_out].set(b2)

    nf16 = [jax.ShapeDtypeStruct((n, LANES), jnp.float32)] * 4
    rows16 = _rows(n, LANES)

    degp = _make_deg(n, nctot)(eidx, zeros, ones)
    xs, selfb1, dis, inv = pl.pallas_call(
        _tc_pre, out_shape=nf16, grid=grid,
        in_specs=[_rows(n, d_in), _full(d_in, LANES), _full(LANES), _part(n)],
        out_specs=[rows16] * 4)(x, W1, b1, degp)
    s1p = _make_agg(n, nctot)(xs, eidx, zeros)
    hs, self2 = pl.pallas_call(
        _tc_mid, out_shape=nf16[:2], grid=grid,
        in_specs=[_part(n), rows16, rows16, rows16,
                  _full(LANES, LANES), _full(LANES)],
        out_specs=[rows16] * 2)(s1p, selfb1, dis, inv, w2p, b2p)
    s2p = _make_agg(n, nctot)(hs, eidx, zeros)
    out = pl.pallas_call(
        functools.partial(_tc_post, d_out=d_out),
        out_shape=jax.ShapeDtypeStruct((n, d_out), jnp.float32),
        grid=grid,
        in_specs=[_part(n), rows16, rows16],
        out_specs=pl.BlockSpec((BN, d_out), lambda i: (i, 0)),
    )(s2p, self2, dis)
    return out


# speedup vs baseline: 1.0175x; 1.0175x over previous
import functools

import jax
import jax.numpy as jnp
from jax import lax
from jax.experimental import pallas as pl
from jax.experimental.pallas import tpu as pltpu
from jax.experimental.pallas import tpu_sc as plsc

NC = 2
NS = 16
NW = NC * NS
LANES = 16
BE = 128
NBUF = 4
PADR = 512


def _row_part(n_nodes):
    rbase = (n_nodes // (NS * 8)) * 8
    rem = n_nodes - rbase * NS
    return rbase, rem


@functools.lru_cache(maxsize=None)
def _make_deg(n_nodes, nctot):
    nch = nctot // NW
    njb = nch // NBUF
    rbase, rem = _row_part(n_nodes)
    mesh = plsc.VectorSubcoreMesh(core_axis_name="c", subcore_axis_name="s")

    @functools.partial(
        pl.kernel,
        mesh=mesh,
        compiler_params=pltpu.CompilerParams(use_tc_tiling_on_sc=False),
        out_type=jax.ShapeDtypeStruct((NC, n_nodes, LANES), jnp.float32),
        scratch_types=[
            pltpu.VMEM((nch, BE), jnp.int32),
            pltpu.VMEM((BE, LANES), jnp.float32),
            pltpu.VMEM_SHARED((n_nodes + PADR, LANES), jnp.float32),
            pltpu.SemaphoreType.DMA((NBUF,)),
        ],
    )
    def deg_kernel(eidx_hbm, zeros_hbm, ones_hbm, out_hbm,
                   dst_v, ones_v, acc, dsem):
        c = lax.axis_index("c")
        s = lax.axis_index("s")
        wid = s * NC + c
        pltpu.sync_copy(zeros_hbm.at[pl.ds(s * rbase, rbase)],
                        acc.at[pl.ds(s * rbase, rbase)])
        if rem:
            @pl.when(s == 0)
            def _():
                pltpu.sync_copy(zeros_hbm.at[pl.ds(rbase * NS, rem)],
                                acc.at[pl.ds(rbase * NS, rem)])
        pltpu.sync_copy(ones_hbm, ones_v)
        pltpu.sync_copy(eidx_hbm.at[1, pl.ds(wid * nch, nch)], dst_v)
        plsc.subcore_barrier()

        def scat(i, b):
            pltpu.async_copy(ones_v, acc.at[dst_v.at[i]], dsem.at[b],
                             add=True)

        def scat_wait(i, b):
            pltpu.make_async_copy(ones_v, acc.at[dst_v.at[i]],
                                  dsem.at[b]).wait()

        for b in range(NBUF):
            scat(b, b)

        def body(j, carry):
            i0 = j * NBUF
            for b in range(NBUF):
                scat_wait(i0 + b, b)
                scat(i0 + NBUF + b, b)
            return carry

        lax.fori_loop(0, njb - 1, body, 0)
        i0 = (njb - 1) * NBUF
        for b in range(NBUF):
            scat_wait(i0 + b, b)
        plsc.subcore_barrier()
        pltpu.sync_copy(acc.at[pl.ds(s * rbase, rbase)],
                        out_hbm.at[c, pl.ds(s * rbase, rbase)])
        if rem:
            @pl.when(s == 0)
            def _():
                pltpu.sync_copy(acc.at[pl.ds(rbase * NS, rem)],
                                out_hbm.at[c, pl.ds(rbase * NS, rem)])

    return deg_kernel


@functools.lru_cache(maxsize=None)
def _make_agg(n_nodes, nctot):
    nch = nctot // NW
    njb = nch // NBUF
    rbase, rem = _row_part(n_nodes)
    mesh = plsc.VectorSubcoreMesh(core_axis_name="c", subcore_axis_name="s")

    @functools.partial(
        pl.kernel,
        mesh=mesh,
        compiler_params=pltpu.CompilerParams(use_tc_tiling_on_sc=False),
        out_type=jax.ShapeDtypeStruct((NC, n_nodes, LANES), jnp.float32),
        scratch_types=[
            pltpu.VMEM((nch, BE), jnp.int32),
            pltpu.VMEM((nch, BE), jnp.int32),
            pltpu.VMEM((NBUF, BE, LANES), jnp.float32),
            pltpu.VMEM_SHARED((n_nodes + PADR, LANES), jnp.float32),
            pltpu.SemaphoreType.DMA((NBUF,)),
            pltpu.SemaphoreType.DMA((NBUF,)),
        ],
    )
    def agg_kernel(table_hbm, eidx_hbm, zeros_hbm, out_hbm,
                   src_v, dst_v, rows_v, acc, gsem, ssem):
        c = lax.axis_index("c")
        s = lax.axis_index("s")
        wid = s * NC + c
        pltpu.sync_copy(zeros_hbm.at[pl.ds(s * rbase, rbase)],
                        acc.at[pl.ds(s * rbase, rbase)])
        if rem:
            @pl.when(s == 0)
            def _():
                pltpu.sync_copy(zeros_hbm.at[pl.ds(rbase * NS, rem)],
                                acc.at[pl.ds(rbase * NS, rem)])
        pltpu.sync_copy(eidx_hbm.at[0, pl.ds(wid * nch, nch)], src_v)
        pltpu.sync_copy(eidx_hbm.at[1, pl.ds(wid * nch, nch)], dst_v)
        plsc.subcore_barrier()

        def gather(i, b):
            pltpu.async_copy(table_hbm.at[src_v.at[i]], rows_v.at[b],
                             gsem.at[b])

        def gather_wait(i, b):
            pltpu.make_async_copy(table_hbm.at[src_v.at[i]], rows_v.at[b],
                                  gsem.at[b]).wait()

        def scat(i, b):
            pltpu.async_copy(rows_v.at[b], acc.at[dst_v.at[i]], ssem.at[b],
                             add=True)

        def scat_wait(i, b):
            pltpu.make_async_copy(rows_v.at[b], acc.at[dst_v.at[i]],
                                  ssem.at[b]).wait()

        for b in range(NBUF):
            gather(b, b)

        def body(j, carry):
            i0 = j * NBUF
            for b in range(NBUF):
                gather_wait(i0 + b, b)
                scat(i0 + b, b)
            for b in range(NBUF):
                scat_wait(i0 + b, b)
                gather(i0 + NBUF + b, b)
            return carry

        lax.fori_loop(0, njb - 1, body, 0)
        i0 = (njb - 1) * NBUF
        for b in range(NBUF):
            gather_wait(i0 + b, b)
            scat(i0 + b, b)
        for b in range(NBUF):
            scat_wait(i0 + b, b)
        plsc.subcore_barrier()
        pltpu.sync_copy(acc.at[pl.ds(s * rbase, rbase)],
                        out_hbm.at[c, pl.ds(s * rbase, rbase)])
        if rem:
            @pl.when(s == 0)
            def _():
                pltpu.sync_copy(acc.at[pl.ds(rbase * NS, rem)],
                                out_hbm.at[c, pl.ds(rbase * NS, rem)])

    return agg_kernel


def _tc_pre(x_ref, w1_ref, b1_ref, degp_ref,
            xs_ref, selfb1_ref, dis_ref, inv_ref):
    xw = jnp.dot(x_ref[...], w1_ref[...], preferred_element_type=jnp.float32)
    deg = degp_ref[0] + degp_ref[1] + 1.0
    dis = lax.rsqrt(deg)
    inv = 1.0 / deg
    xs_ref[...] = xw * dis
    selfb1_ref[...] = xw * inv + b1_ref[...]
    dis_ref[...] = dis
    inv_ref[...] = inv


def _tc_mid(s1p_ref, selfb1_ref, dis_ref, inv_ref, w2p_ref, b2p_ref,
            hs_ref, self2_ref):
    s1 = s1p_ref[0] + s1p_ref[1]
    h = jnp.maximum(dis_ref[...] * s1 + selfb1_ref[...], 0.0)
    hw = jnp.dot(h, w2p_ref[...], preferred_element_type=jnp.float32)
    hs_ref[...] = hw * dis_ref[...]
    self2_ref[...] = hw * inv_ref[...] + b2p_ref[...]


def _tc_post(s2p_ref, self2_ref, dis_ref, out_ref, *, d_out):
    o = dis_ref[...] * (s2p_ref[0] + s2p_ref[1]) + self2_ref[...]
    col = lax.broadcasted_iota(jnp.int32, o.shape, 1)
    om = jnp.where(col < d_out, o, -jnp.inf)
    m = jnp.max(om, axis=1, keepdims=True)
    e = jnp.exp(om - m)
    ssum = jnp.sum(e, axis=1, keepdims=True)
    out_ref[...] = (o - m - jnp.log(ssum))[:, :d_out]


BN = 1000


def _rows(n, width):
    return pl.BlockSpec((BN, width), lambda i: (i, 0))


def _part(n):
    return pl.BlockSpec((NC, BN, LANES), lambda i: (0, i, 0))


def _full(*shape):
    return pl.BlockSpec(shape, lambda i: (0,) * len(shape))


def kernel(x, edge_index, W1, b1, W2, b2):
    n, d_in = x.shape
    d_hid = W1.shape[1]
    d_out = W2.shape[1]
    n_edges = edge_index.shape[1]
    assert d_hid == LANES and d_out <= LANES
    assert n % NS == 0 and n % BN == 0
    grid = (n // BN,)

    nctot = -(-n_edges // (NW * BE * NBUF)) * NW * NBUF
    e_pad = nctot * BE - n_edges
    pad = jnp.concatenate(
        [jnp.zeros((1, e_pad), jnp.int32),
         (n + jnp.arange(e_pad, dtype=jnp.int32) % PADR)[None, :]], axis=0)
    eidx = jnp.concatenate([edge_index, pad], axis=1).reshape(2, nctot, BE)

    zeros = jnp.zeros((n, LANES), jnp.float32)
    ones = jnp.ones((BE, LANES), jnp.float32)
    w2p = jnp.zeros((LANES, LANES), jnp.float32).at[:, :d_out].set(W2)
    b2p = jnp.zeros((LANES,), jnp.float32).at[:d_out].set(b2)

    nf16 = [jax.ShapeDtypeStruct((n, LANES), jnp.float32)] * 4
    rows16 = _rows(n, LANES)

    degp = _make_deg(n, nctot)(eidx, zeros, ones)
    xs, selfb1, dis, inv = pl.pallas_call(
        _tc_pre, out_shape=nf16, grid=grid,
        in_specs=[_rows(n, d_in), _full(d_in, LANES), _full(LANES), _part(n)],
        out_specs=[rows16] * 4)(x, W1, b1, degp)
    s1p = _make_agg(n, nctot)(xs, eidx, zeros)
    hs, self2 = pl.pallas_call(
        _tc_mid, out_shape=nf16[:2], grid=grid,
        in_specs=[_part(n), rows16, rows16, rows16,
                  _full(LANES, LANES), _full(LANES)],
        out_specs=[rows16] * 2)(s1p, selfb1, dis, inv, w2p, b2p)
    s2p = _make_agg(n, nctot)(hs, eidx, zeros)
    out = pl.pallas_call(
        functools.partial(_tc_post, d_out=d_out),
        out_shape=jax.ShapeDtypeStruct((n, d_out), jnp.float32),
        grid=grid,
        in_specs=[_part(n), rows16, rows16],
        out_specs=pl.BlockSpec((BN, d_out), lambda i: (i, 0)),
    )(s2p, self2, dis)
    return out


# speedup vs baseline: 1.5988x; 1.5714x over previous
import functools

import jax
import jax.numpy as jnp
from jax import lax
from jax.experimental import pallas as pl
from jax.experimental.pallas import tpu as pltpu
from jax.experimental.pallas import tpu_sc as plsc

NC = 2
NS = 16
NW = NC * NS
LANES = 16
BE = 128
NBUF = 4
PADR = 512


def _row_part(n_nodes):
    rbase = (n_nodes // (NS * 8)) * 8
    rem = n_nodes - rbase * NS
    return rbase, rem


@functools.lru_cache(maxsize=None)
def _make_deg(n_nodes, nctot):
    nch = nctot // NW
    njb = nch // NBUF
    rbase, rem = _row_part(n_nodes)
    mesh = plsc.VectorSubcoreMesh(core_axis_name="c", subcore_axis_name="s")

    @functools.partial(
        pl.kernel,
        mesh=mesh,
        compiler_params=pltpu.CompilerParams(use_tc_tiling_on_sc=False),
        out_type=jax.ShapeDtypeStruct((NC, n_nodes, LANES), jnp.float32),
        scratch_types=[
            pltpu.VMEM((nch, BE), jnp.int32),
            pltpu.VMEM((BE, LANES), jnp.float32),
            pltpu.VMEM_SHARED((n_nodes + PADR, LANES), jnp.float32),
            pltpu.SemaphoreType.DMA((NBUF,)),
        ],
    )
    def deg_kernel(eidx_hbm, zeros_hbm, ones_hbm, out_hbm,
                   dst_v, ones_v, acc, dsem):
        c = lax.axis_index("c")
        s = lax.axis_index("s")
        wid = s * NC + c
        pltpu.sync_copy(zeros_hbm.at[pl.ds(s * rbase, rbase)],
                        acc.at[pl.ds(s * rbase, rbase)])
        if rem:
            @pl.when(s == 0)
            def _():
                pltpu.sync_copy(zeros_hbm.at[pl.ds(rbase * NS, rem)],
                                acc.at[pl.ds(rbase * NS, rem)])
        pltpu.sync_copy(ones_hbm, ones_v)
        pltpu.sync_copy(eidx_hbm.at[1, pl.ds(wid * nch, nch)], dst_v)
        plsc.subcore_barrier()

        def scat(i, b):
            pltpu.async_copy(ones_v, acc.at[dst_v.at[i]], dsem.at[b],
                             add=True)

        def scat_wait(i, b):
            pltpu.make_async_copy(ones_v, acc.at[dst_v.at[i]],
                                  dsem.at[b]).wait()

        for b in range(NBUF):
            scat(b, b)

        def body(j, carry):
            i0 = j * NBUF
            for b in range(NBUF):
                scat_wait(i0 + b, b)
                scat(i0 + NBUF + b, b)
            return carry

        lax.fori_loop(0, njb - 1, body, 0)
        i0 = (njb - 1) * NBUF
        for b in range(NBUF):
            scat_wait(i0 + b, b)
        plsc.subcore_barrier()
        pltpu.sync_copy(acc.at[pl.ds(s * rbase, rbase)],
                        out_hbm.at[c, pl.ds(s * rbase, rbase)])
        if rem:
            @pl.when(s == 0)
            def _():
                pltpu.sync_copy(acc.at[pl.ds(rbase * NS, rem)],
                                out_hbm.at[c, pl.ds(rbase * NS, rem)])

    return deg_kernel


@functools.lru_cache(maxsize=None)
def _make_agg(n_nodes, nctot):
    nch = nctot // NW
    njb = nch // NBUF
    rbase, rem = _row_part(n_nodes)
    mesh = plsc.VectorSubcoreMesh(core_axis_name="c", subcore_axis_name="s")

    @functools.partial(
        pl.kernel,
        mesh=mesh,
        compiler_params=pltpu.CompilerParams(use_tc_tiling_on_sc=False),
        out_type=jax.ShapeDtypeStruct((NC, n_nodes, LANES), jnp.float32),
        scratch_types=[
            pltpu.VMEM((nch, BE), jnp.int32),
            pltpu.VMEM((nch, BE), jnp.int32),
            pltpu.VMEM((NBUF, BE, LANES), jnp.float32),
            pltpu.VMEM_SHARED((n_nodes + PADR, LANES), jnp.float32),
            pltpu.SemaphoreType.DMA((NBUF,)),
            pltpu.SemaphoreType.DMA((NBUF,)),
        ],
    )
    def agg_kernel(table_hbm, eidx_hbm, zeros_hbm, out_hbm,
                   src_v, dst_v, rows_v, acc, gsem, ssem):
        c = lax.axis_index("c")
        s = lax.axis_index("s")
        wid = s * NC + c
        pltpu.sync_copy(zeros_hbm.at[pl.ds(s * rbase, rbase)],
                        acc.at[pl.ds(s * rbase, rbase)])
        if rem:
            @pl.when(s == 0)
            def _():
                pltpu.sync_copy(zeros_hbm.at[pl.ds(rbase * NS, rem)],
                                acc.at[pl.ds(rbase * NS, rem)])
        pltpu.sync_copy(eidx_hbm.at[0, pl.ds(wid * nch, nch)], src_v)
        pltpu.sync_copy(eidx_hbm.at[1, pl.ds(wid * nch, nch)], dst_v)
        plsc.subcore_barrier()

        def gather(i, b):
            pltpu.async_copy(table_hbm.at[src_v.at[i]], rows_v.at[b],
                             gsem.at[b])

        def gather_wait(i, b):
            pltpu.make_async_copy(table_hbm.at[src_v.at[i]], rows_v.at[b],
                                  gsem.at[b]).wait()

        def scat(i, b):
            pltpu.async_copy(rows_v.at[b], acc.at[dst_v.at[i]], ssem.at[b],
                             add=True)

        def scat_wait(i, b):
            pltpu.make_async_copy(rows_v.at[b], acc.at[dst_v.at[i]],
                                  ssem.at[b]).wait()

        for b in range(NBUF):
            gather(b, b)

        def body(j, carry):
            i0 = j * NBUF
            for b in range(NBUF):
                gather_wait(i0 + b, b)
                scat(i0 + b, b)
            for b in range(NBUF):
                scat_wait(i0 + b, b)
                gather(i0 + NBUF + b, b)
            return carry

        lax.fori_loop(0, njb - 1, body, 0)
        i0 = (njb - 1) * NBUF
        for b in range(NBUF):
            gather_wait(i0 + b, b)
            scat(i0 + b, b)
        for b in range(NBUF):
            scat_wait(i0 + b, b)
        plsc.subcore_barrier()
        pltpu.sync_copy(acc.at[pl.ds(s * rbase, rbase)],
                        out_hbm.at[c, pl.ds(s * rbase, rbase)])
        if rem:
            @pl.when(s == 0)
            def _():
                pltpu.sync_copy(acc.at[pl.ds(rbase * NS, rem)],
                                out_hbm.at[c, pl.ds(rbase * NS, rem)])

    return agg_kernel


def _tc_pre(x_ref, w1_ref, b1_ref, degp_ref,
            xs_ref, selfb1_ref, dis_ref, inv_ref):
    xw = jnp.dot(x_ref[...], w1_ref[...], preferred_element_type=jnp.float32)
    deg = degp_ref[0] + degp_ref[1] + 1.0
    dis = lax.rsqrt(deg)
    inv = 1.0 / deg
    xs_ref[...] = xw * dis
    selfb1_ref[...] = xw * inv + b1_ref[...]
    dis_ref[...] = dis
    inv_ref[...] = inv


def _tc_mid(s1p_ref, selfb1_ref, dis_ref, inv_ref, w2p_ref, b2p_ref,
            hs_ref, self2_ref):
    s1 = s1p_ref[0] + s1p_ref[1]
    h = jnp.maximum(dis_ref[...] * s1 + selfb1_ref[...], 0.0)
    hw = jnp.dot(h, w2p_ref[...], preferred_element_type=jnp.float32)
    hs_ref[...] = hw * dis_ref[...]
    self2_ref[...] = hw * inv_ref[...] + b2p_ref[...]


def _tc_post(s2p_ref, self2_ref, dis_ref, out_ref, *, d_out):
    o = dis_ref[...] * (s2p_ref[0] + s2p_ref[1]) + self2_ref[...]
    col = lax.broadcasted_iota(jnp.int32, o.shape, 1)
    om = jnp.where(col < d_out, o, -jnp.inf)
    m = jnp.max(om, axis=1, keepdims=True)
    e = jnp.exp(om - m)
    ssum = jnp.sum(e, axis=1, keepdims=True)
    out_ref[...] = (o - m - jnp.log(ssum))[:, :d_out]


BN = 1000


def _rows(n, width):
    return pl.BlockSpec((BN, width), lambda i: (i, 0))


def _part(n):
    return pl.BlockSpec((NC, BN, LANES), lambda i: (0, i, 0))


def _full(*shape):
    return pl.BlockSpec(shape, lambda i: (0,) * len(shape))


def kernel(x, edge_index, W1, b1, W2, b2):
    n, d_in = x.shape
    d_hid = W1.shape[1]
    d_out = W2.shape[1]
    n_edges = edge_index.shape[1]
    assert d_hid == LANES and d_out <= LANES
    assert n % NS == 0 and n % BN == 0
    grid = (n // BN,)

    nctot = -(-n_edges // (NW * BE * NBUF)) * NW * NBUF
    e_pad = nctot * BE - n_edges
    spread = jnp.arange(e_pad, dtype=jnp.int32)
    pad = jnp.concatenate(
        [(spread * 61 % n)[None, :],
         (n + spread % PADR)[None, :]], axis=0)
    eidx = jnp.concatenate([edge_index, pad], axis=1).reshape(2, nctot, BE)

    zeros = jnp.zeros((n, LANES), jnp.float32)
    ones = jnp.ones((BE, LANES), jnp.float32)
    w2p = jnp.zeros((LANES, LANES), jnp.float32).at[:, :d_out].set(W2)
    b2p = jnp.zeros((LANES,), jnp.float32).at[:d_out].set(b2)

    nf16 = [jax.ShapeDtypeStruct((n, LANES), jnp.float32)] * 4
    rows16 = _rows(n, LANES)

    degp = _make_deg(n, nctot)(eidx, zeros, ones)
    xs, selfb1, dis, inv = pl.pallas_call(
        _tc_pre, out_shape=nf16, grid=grid,
        in_specs=[_rows(n, d_in), _full(d_in, LANES), _full(LANES), _part(n)],
        out_specs=[rows16] * 4)(x, W1, b1, degp)
    s1p = _make_agg(n, nctot)(xs, eidx, zeros)
    hs, self2 = pl.pallas_call(
        _tc_mid, out_shape=nf16[:2], grid=grid,
        in_specs=[_part(n), rows16, rows16, rows16,
                  _full(LANES, LANES), _full(LANES)],
        out_specs=[rows16] * 2)(s1p, selfb1, dis, inv, w2p, b2p)
    s2p = _make_agg(n, nctot)(hs, eidx, zeros)
    out = pl.pallas_call(
        functools.partial(_tc_post, d_out=d_out),
        out_shape=jax.ShapeDtypeStruct((n, d_out), jnp.float32),
        grid=grid,
        in_specs=[_part(n), rows16, rows16],
        out_specs=pl.BlockSpec((BN, d_out), lambda i: (i, 0)),
    )(s2p, self2, dis)
    return out


# speedup vs baseline: 1.7864x; 1.1173x over previous
import functools

import jax
import jax.numpy as jnp
from jax import lax
from jax.experimental import pallas as pl
from jax.experimental.pallas import tpu as pltpu
from jax.experimental.pallas import tpu_sc as plsc

NC = 2
NS = 16
NW = NC * NS
LANES = 16
BE = 128
NBUF = 8
PADR = 512


def _row_part(n_nodes):
    rbase = (n_nodes // (NS * 8)) * 8
    rem = n_nodes - rbase * NS
    return rbase, rem


@functools.lru_cache(maxsize=None)
def _make_deg(n_nodes, nctot):
    nch = nctot // NW
    njb = nch // NBUF
    rbase, rem = _row_part(n_nodes)
    mesh = plsc.VectorSubcoreMesh(core_axis_name="c", subcore_axis_name="s")

    @functools.partial(
        pl.kernel,
        mesh=mesh,
        compiler_params=pltpu.CompilerParams(use_tc_tiling_on_sc=False),
        out_type=jax.ShapeDtypeStruct((NC, n_nodes, LANES), jnp.float32),
        scratch_types=[
            pltpu.VMEM((nch, BE), jnp.int32),
            pltpu.VMEM((BE, LANES), jnp.float32),
            pltpu.VMEM_SHARED((n_nodes + PADR, LANES), jnp.float32),
            pltpu.SemaphoreType.DMA((NBUF,)),
        ],
    )
    def deg_kernel(eidx_hbm, zeros_hbm, ones_hbm, out_hbm,
                   dst_v, ones_v, acc, dsem):
        c = lax.axis_index("c")
        s = lax.axis_index("s")
        wid = s * NC + c
        pltpu.sync_copy(zeros_hbm.at[pl.ds(s * rbase, rbase)],
                        acc.at[pl.ds(s * rbase, rbase)])
        if rem:
            @pl.when(s == 0)
            def _():
                pltpu.sync_copy(zeros_hbm.at[pl.ds(rbase * NS, rem)],
                                acc.at[pl.ds(rbase * NS, rem)])
        pltpu.sync_copy(ones_hbm, ones_v)
        pltpu.sync_copy(eidx_hbm.at[1, pl.ds(wid * nch, nch)], dst_v)
        plsc.subcore_barrier()

        def scat(i, b):
            pltpu.async_copy(ones_v, acc.at[dst_v.at[i]], dsem.at[b],
                             add=True)

        def scat_wait(i, b):
            pltpu.make_async_copy(ones_v, acc.at[dst_v.at[i]],
                                  dsem.at[b]).wait()

        for b in range(NBUF):
            scat(b, b)

        def body(j, carry):
            i0 = j * NBUF
            for b in range(NBUF):
                scat_wait(i0 + b, b)
                scat(i0 + NBUF + b, b)
            return carry

        lax.fori_loop(0, njb - 1, body, 0)
        i0 = (njb - 1) * NBUF
        for b in range(NBUF):
            scat_wait(i0 + b, b)
        plsc.subcore_barrier()
        pltpu.sync_copy(acc.at[pl.ds(s * rbase, rbase)],
                        out_hbm.at[c, pl.ds(s * rbase, rbase)])
        if rem:
            @pl.when(s == 0)
            def _():
                pltpu.sync_copy(acc.at[pl.ds(rbase * NS, rem)],
                                out_hbm.at[c, pl.ds(rbase * NS, rem)])

    return deg_kernel


@functools.lru_cache(maxsize=None)
def _make_agg(n_nodes, nctot):
    nch = nctot // NW
    njb = nch // NBUF
    rbase, rem = _row_part(n_nodes)
    mesh = plsc.VectorSubcoreMesh(core_axis_name="c", subcore_axis_name="s")

    @functools.partial(
        pl.kernel,
        mesh=mesh,
        compiler_params=pltpu.CompilerParams(use_tc_tiling_on_sc=False),
        out_type=jax.ShapeDtypeStruct((NC, n_nodes, LANES), jnp.float32),
        scratch_types=[
            pltpu.VMEM((nch, BE), jnp.int32),
            pltpu.VMEM((nch, BE), jnp.int32),
            pltpu.VMEM((NBUF, BE, LANES), jnp.float32),
            pltpu.VMEM_SHARED((n_nodes + PADR, LANES), jnp.float32),
            pltpu.SemaphoreType.DMA((NBUF,)),
            pltpu.SemaphoreType.DMA((NBUF,)),
        ],
    )
    def agg_kernel(table_hbm, eidx_hbm, zeros_hbm, out_hbm,
                   src_v, dst_v, rows_v, acc, gsem, ssem):
        c = lax.axis_index("c")
        s = lax.axis_index("s")
        wid = s * NC + c
        pltpu.sync_copy(zeros_hbm.at[pl.ds(s * rbase, rbase)],
                        acc.at[pl.ds(s * rbase, rbase)])
        if rem:
            @pl.when(s == 0)
            def _():
                pltpu.sync_copy(zeros_hbm.at[pl.ds(rbase * NS, rem)],
                                acc.at[pl.ds(rbase * NS, rem)])
        pltpu.sync_copy(eidx_hbm.at[0, pl.ds(wid * nch, nch)], src_v)
        pltpu.sync_copy(eidx_hbm.at[1, pl.ds(wid * nch, nch)], dst_v)
        plsc.subcore_barrier()

        def gather(i, b):
            pltpu.async_copy(table_hbm.at[src_v.at[i]], rows_v.at[b],
                             gsem.at[b])

        def gather_wait(i, b):
            pltpu.make_async_copy(table_hbm.at[src_v.at[i]], rows_v.at[b],
                                  gsem.at[b]).wait()

        def scat(i, b):
            pltpu.async_copy(rows_v.at[b], acc.at[dst_v.at[i]], ssem.at[b],
                             add=True)

        def scat_wait(i, b):
            pltpu.make_async_copy(rows_v.at[b], acc.at[dst_v.at[i]],
                                  ssem.at[b]).wait()

        for b in range(NBUF):
            gather(b, b)

        def body(j, carry):
            i0 = j * NBUF
            for b in range(NBUF):
                gather_wait(i0 + b, b)
                scat(i0 + b, b)
            for b in range(NBUF):
                scat_wait(i0 + b, b)
                gather(i0 + NBUF + b, b)
            return carry

        lax.fori_loop(0, njb - 1, body, 0)
        i0 = (njb - 1) * NBUF
        for b in range(NBUF):
            gather_wait(i0 + b, b)
            scat(i0 + b, b)
        for b in range(NBUF):
            scat_wait(i0 + b, b)
        plsc.subcore_barrier()
        pltpu.sync_copy(acc.at[pl.ds(s * rbase, rbase)],
                        out_hbm.at[c, pl.ds(s * rbase, rbase)])
        if rem:
            @pl.when(s == 0)
            def _():
                pltpu.sync_copy(acc.at[pl.ds(rbase * NS, rem)],
                                out_hbm.at[c, pl.ds(rbase * NS, rem)])

    return agg_kernel


def _tc_pre(x_ref, w1_ref, b1_ref, degp_ref,
            xs_ref, selfb1_ref, dis_ref, inv_ref):
    xw = jnp.dot(x_ref[...], w1_ref[...], preferred_element_type=jnp.float32)
    deg = degp_ref[0] + degp_ref[1] + 1.0
    dis = lax.rsqrt(deg)
    inv = 1.0 / deg
    xs_ref[...] = xw * dis
    selfb1_ref[...] = xw * inv + b1_ref[...]
    dis_ref[...] = dis
    inv_ref[...] = inv


def _tc_mid(s1p_ref, selfb1_ref, dis_ref, inv_ref, w2p_ref, b2p_ref,
            hs_ref, self2_ref):
    s1 = s1p_ref[0] + s1p_ref[1]
    h = jnp.maximum(dis_ref[...] * s1 + selfb1_ref[...], 0.0)
    hw = jnp.dot(h, w2p_ref[...], preferred_element_type=jnp.float32)
    hs_ref[...] = hw * dis_ref[...]
    self2_ref[...] = hw * inv_ref[...] + b2p_ref[...]


def _tc_post(s2p_ref, self2_ref, dis_ref, out_ref, *, d_out):
    o = dis_ref[...] * (s2p_ref[0] + s2p_ref[1]) + self2_ref[...]
    col = lax.broadcasted_iota(jnp.int32, o.shape, 1)
    om = jnp.where(col < d_out, o, -jnp.inf)
    m = jnp.max(om, axis=1, keepdims=True)
    e = jnp.exp(om - m)
    ssum = jnp.sum(e, axis=1, keepdims=True)
    out_ref[...] = (o - m - jnp.log(ssum))[:, :d_out]


BN = 2000


def _rows(n, width):
    return pl.BlockSpec((BN, width), lambda i: (i, 0))


def _part(n):
    return pl.BlockSpec((NC, BN, LANES), lambda i: (0, i, 0))


def _full(*shape):
    return pl.BlockSpec(shape, lambda i: (0,) * len(shape))


def kernel(x, edge_index, W1, b1, W2, b2):
    n, d_in = x.shape
    d_hid = W1.shape[1]
    d_out = W2.shape[1]
    n_edges = edge_index.shape[1]
    assert d_hid == LANES and d_out <= LANES
    assert n % NS == 0 and n % BN == 0
    grid = (n // BN,)

    nctot = -(-n_edges // (NW * BE * NBUF)) * NW * NBUF
    e_pad = nctot * BE - n_edges
    spread = jnp.arange(e_pad, dtype=jnp.int32)
    pad = jnp.concatenate(
        [(spread * 61 % n)[None, :],
         (n + spread % PADR)[None, :]], axis=0)
    eidx = jnp.concatenate([edge_index, pad], axis=1).reshape(2, nctot, BE)

    zeros = jnp.zeros((n, LANES), jnp.float32)
    ones = jnp.ones((BE, LANES), jnp.float32)
    w2p = jnp.zeros((LANES, LANES), jnp.float32).at[:, :d_out].set(W2)
    b2p = jnp.zeros((LANES,), jnp.float32).at[:d_out].set(b2)

    nf16 = [jax.ShapeDtypeStruct((n, LANES), jnp.float32)] * 4
    rows16 = _rows(n, LANES)

    degp = _make_deg(n, nctot)(eidx, zeros, ones)
    xs, selfb1, dis, inv = pl.pallas_call(
        _tc_pre, out_shape=nf16, grid=grid,
        in_specs=[_rows(n, d_in), _full(d_in, LANES), _full(LANES), _part(n)],
        out_specs=[rows16] * 4)(x, W1, b1, degp)
    s1p = _make_agg(n, nctot)(xs, eidx, zeros)
    hs, self2 = pl.pallas_call(
        _tc_mid, out_shape=nf16[:2], grid=grid,
        in_specs=[_part(n), rows16, rows16, rows16,
                  _full(LANES, LANES), _full(LANES)],
        out_specs=[rows16] * 2)(s1p, selfb1, dis, inv, w2p, b2p)
    s2p = _make_agg(n, nctot)(hs, eidx, zeros)
    out = pl.pallas_call(
        functools.partial(_tc_post, d_out=d_out),
        out_shape=jax.ShapeDtypeStruct((n, d_out), jnp.float32),
        grid=grid,
        in_specs=[_part(n), rows16, rows16],
        out_specs=pl.BlockSpec((BN, d_out), lambda i: (i, 0)),
    )(s2p, self2, dis)
    return out
